# Initial kernel scaffold; baseline (speedup 1.0000x reference)
import jax
import jax.numpy as jnp
from jax.experimental import pallas as pl


def kernel(x, edge_index, W1, b1, W2, b2):
    raise NotImplementedError("write your pallas kernel here")


# speedup vs baseline: 8.0854x; 8.0854x over previous
import functools

import jax
import jax.numpy as jnp
from jax import lax
from jax.experimental import pallas as pl
from jax.experimental.pallas import tpu as pltpu
from jax.experimental.pallas import tpu_sc as plsc

N_NODES = 10000
N_PAD = 10240
N_EDGES = 320000
D_IN = 128
N_CLASSES = 64

NC = 2
NS = 16
EDGES_PER_TILE = N_EDGES // NS
CHUNK = 80
NCHUNKS = EDGES_PER_TILE // CHUNK
ROWS_PER_TILE = N_PAD // NS

_MESH = plsc.VectorSubcoreMesh(
    core_axis_name="c", subcore_axis_name="s", num_cores=NC, num_subcores=NS)

_SC_PARAMS = pltpu.CompilerParams(use_tc_tiling_on_sc=False)


@functools.partial(
    pl.kernel,
    out_type=jax.ShapeDtypeStruct((NC, N_PAD, 16), jnp.float32),
    mesh=_MESH,
    scratch_types=[
        pltpu.VMEM_SHARED((N_PAD, 16), jnp.float32),
        pltpu.VMEM((NCHUNKS, CHUNK), jnp.int32),
        pltpu.VMEM((CHUNK, 16), jnp.float32),
    ],
    compiler_params=_SC_PARAMS,
)
def _sc_degrees(edges_hbm, ones_hbm, zeros_hbm, out_hbm, hist_sp, idx_v, ones_v):
    c = lax.axis_index("c")
    s = lax.axis_index("s")
    rows = pl.ds(s * ROWS_PER_TILE, ROWS_PER_TILE)
    pltpu.sync_copy(zeros_hbm.at[rows], hist_sp.at[rows])
    pltpu.sync_copy(edges_hbm.at[c, s], idx_v)
    pltpu.sync_copy(ones_hbm, ones_v)
    plsc.subcore_barrier()

    def body(j, carry):
        pltpu.sync_copy(ones_v, hist_sp.at[idx_v.at[j]], add=True)
        return carry

    lax.fori_loop(0, NCHUNKS, body, 0)
    plsc.subcore_barrier()
    pltpu.sync_copy(hist_sp.at[rows], out_hbm.at[c, rows])


def _make_sc_spmm(dh):
    @functools.partial(
        pl.kernel,
        out_type=jax.ShapeDtypeStruct((NC, N_PAD, dh), jnp.float32),
        mesh=_MESH,
        scratch_types=[
            pltpu.VMEM_SHARED((N_PAD, dh), jnp.float32),
            pltpu.VMEM_SHARED((N_PAD, dh), jnp.float32),
            pltpu.VMEM((NCHUNKS, CHUNK), jnp.int32),
            pltpu.VMEM((NCHUNKS, CHUNK), jnp.int32),
            pltpu.VMEM((CHUNK, dh), jnp.float32),
            pltpu.SemaphoreType.DMA,
        ],
        compiler_params=_SC_PARAMS,
    )
    def _sc_spmm(u_hbm, src_hbm, dst_hbm, zeros_hbm, out_hbm,
                 u_sp, agg_sp, src_v, dst_v, gbuf, sem):
        c = lax.axis_index("c")
        s = lax.axis_index("s")
        rows = pl.ds(s * ROWS_PER_TILE, ROWS_PER_TILE)
        pltpu.sync_copy(u_hbm.at[c, rows], u_sp.at[rows])
        pltpu.sync_copy(zeros_hbm.at[rows], agg_sp.at[rows])
        pltpu.sync_copy(src_hbm.at[s], src_v)
        pltpu.sync_copy(dst_hbm.at[s], dst_v)
        plsc.subcore_barrier()

        def body(j, carry):
            pltpu.async_copy(u_sp.at[src_v.at[j]], gbuf, sem).wait()
            pltpu.sync_copy(gbuf, agg_sp.at[dst_v.at[j]], add=True)
            return carry

        lax.fori_loop(0, NCHUNKS, body, 0)
        plsc.subcore_barrier()
        pltpu.sync_copy(agg_sp.at[rows], out_hbm.at[c, rows])

    return _sc_spmm


_sc_spmm_64 = _make_sc_spmm(64)
_sc_spmm_32 = _make_sc_spmm(32)


_BLK = 1024
_GRID = N_PAD // _BLK


def _norm_col(deg_ref):
    return lax.rsqrt(jnp.clip(deg_ref[:, :1], 1.0, None))


def _tc1_body(x_ref, w1_ref, dego_ref, out_ref):
    ns = _norm_col(dego_ref)
    u = jnp.dot(x_ref[...], w1_ref[...], preferred_element_type=jnp.float32) * ns
    out_ref[0] = u[:, :64]
    out_ref[1] = u[:, 64:]


def _tc2_body(agg_ref, degi_ref, dego_ref, w2_ref, b1_ref, out_ref):
    nd = _norm_col(degi_ref)
    ns = _norm_col(dego_ref)
    agg = jnp.concatenate([agg_ref[0], agg_ref[1]], axis=1)
    h = jax.nn.relu(agg * nd + b1_ref[...])
    u2 = jnp.dot(h, w2_ref[...], preferred_element_type=jnp.float32) * ns
    out_ref[0] = u2[:, :32]
    out_ref[1] = u2[:, 32:]


def _tc3_body(agg_ref, degi_ref, b2_ref, out_ref):
    nd = _norm_col(degi_ref)
    agg = jnp.concatenate([agg_ref[0], agg_ref[1]], axis=1)
    out_ref[...] = jax.nn.relu(agg * nd + b2_ref[...])


_tc1 = pl.pallas_call(
    _tc1_body,
    grid=(_GRID,),
    in_specs=[
        pl.BlockSpec((_BLK, D_IN), lambda i: (i, 0)),
        pl.BlockSpec((D_IN, D_IN), lambda i: (0, 0)),
        pl.BlockSpec((_BLK, 16), lambda i: (i, 0)),
    ],
    out_specs=pl.BlockSpec((NC, _BLK, 64), lambda i: (0, i, 0)),
    out_shape=jax.ShapeDtypeStruct((NC, N_PAD, 64), jnp.float32),
)

_tc2 = pl.pallas_call(
    _tc2_body,
    grid=(_GRID,),
    in_specs=[
        pl.BlockSpec((NC, _BLK, 64), lambda i: (0, i, 0)),
        pl.BlockSpec((_BLK, 16), lambda i: (i, 0)),
        pl.BlockSpec((_BLK, 16), lambda i: (i, 0)),
        pl.BlockSpec((D_IN, N_CLASSES), lambda i: (0, 0)),
        pl.BlockSpec((1, D_IN), lambda i: (0, 0)),
    ],
    out_specs=pl.BlockSpec((NC, _BLK, 32), lambda i: (0, i, 0)),
    out_shape=jax.ShapeDtypeStruct((NC, N_PAD, 32), jnp.float32),
)

_tc3 = pl.pallas_call(
    _tc3_body,
    grid=(_GRID,),
    in_specs=[
        pl.BlockSpec((NC, _BLK, 32), lambda i: (0, i, 0)),
        pl.BlockSpec((_BLK, 16), lambda i: (i, 0)),
        pl.BlockSpec((1, N_CLASSES), lambda i: (0, 0)),
    ],
    out_specs=pl.BlockSpec((_BLK, N_CLASSES), lambda i: (i, 0)),
    out_shape=jax.ShapeDtypeStruct((N_PAD, N_CLASSES), jnp.float32),
)


def kernel(x, edge_index, W1, b1, W2, b2):
    edge_index = edge_index.astype(jnp.int32)
    edges_r = edge_index.reshape(NC, NS, NCHUNKS, CHUNK)
    src_r = edges_r[0]
    dst_r = edges_r[1]

    x_pad = jnp.pad(x, ((0, N_PAD - N_NODES), (0, 0)))
    ones16 = jnp.ones((CHUNK, 16), jnp.float32)
    z16 = jnp.zeros((N_PAD, 16), jnp.float32)
    z64 = jnp.zeros((N_PAD, 64), jnp.float32)
    z32 = jnp.zeros((N_PAD, 32), jnp.float32)

    degs = _sc_degrees(edges_r, ones16, z16)
    deg_out = degs[0]
    deg_in = degs[1]

    u1 = _tc1(x_pad, W1, deg_out)
    agg1 = _sc_spmm_64(u1, src_r, dst_r, z64)
    u2 = _tc2(agg1, deg_in, deg_out, W2, b1.reshape(1, -1))
    agg2 = _sc_spmm_32(u2, src_r, dst_r, z32)
    out = _tc3(agg2, deg_in, b2.reshape(1, -1))
    return out[:N_NODES]


# speedup vs baseline: 12.1771x; 1.5061x over previous
import functools

import jax
import jax.numpy as jnp
from jax import lax
from jax.experimental import pallas as pl
from jax.experimental.pallas import tpu as pltpu
from jax.experimental.pallas import tpu_sc as plsc

N_NODES = 10000
N_PAD = 10240
N_EDGES = 320000
D_IN = 128
N_CLASSES = 64

NC = 2
NS = 16
EDGES_PER_TILE = N_EDGES // NS
CHUNK = 80
NCHUNKS = EDGES_PER_TILE // CHUNK
ROWS_PER_TILE = N_PAD // NS

_MESH = plsc.VectorSubcoreMesh(
    core_axis_name="c", subcore_axis_name="s", num_cores=NC, num_subcores=NS)

_SC_PARAMS = pltpu.CompilerParams(use_tc_tiling_on_sc=False)


@functools.partial(
    pl.kernel,
    out_type=jax.ShapeDtypeStruct((NC, N_PAD, 16), jnp.float32),
    mesh=_MESH,
    scratch_types=[
        pltpu.VMEM_SHARED((N_PAD, 16), jnp.float32),
        pltpu.VMEM((NCHUNKS, CHUNK), jnp.int32),
        pltpu.VMEM((CHUNK, 16), jnp.float32),
    ],
    compiler_params=_SC_PARAMS,
)
def _sc_degrees(edges_hbm, ones_hbm, zeros_hbm, out_hbm, hist_sp, idx_v, ones_v):
    c = lax.axis_index("c")
    s = lax.axis_index("s")
    rows = pl.ds(s * ROWS_PER_TILE, ROWS_PER_TILE)
    pltpu.sync_copy(zeros_hbm.at[rows], hist_sp.at[rows])
    pltpu.sync_copy(edges_hbm.at[c, s], idx_v)
    pltpu.sync_copy(ones_hbm, ones_v)
    plsc.subcore_barrier()

    def body(j, carry):
        pltpu.sync_copy(ones_v, hist_sp.at[idx_v.at[j]], add=True)
        return carry

    lax.fori_loop(0, NCHUNKS, body, 0)
    plsc.subcore_barrier()
    pltpu.sync_copy(hist_sp.at[rows], out_hbm.at[c, rows])


NBUF = 5


def _make_sc_spmm(dh):
    @functools.partial(
        pl.kernel,
        out_type=jax.ShapeDtypeStruct((NC, N_PAD, dh), jnp.float32),
        mesh=_MESH,
        scratch_types=[
            pltpu.VMEM_SHARED((N_PAD, dh), jnp.float32),
            pltpu.VMEM((NCHUNKS, CHUNK), jnp.int32),
            pltpu.VMEM((NCHUNKS, CHUNK), jnp.int32),
            [pltpu.VMEM((CHUNK, dh), jnp.float32)] * NBUF,
            [pltpu.SemaphoreType.DMA] * NBUF,
            [pltpu.SemaphoreType.DMA] * NBUF,
        ],
        compiler_params=_SC_PARAMS,
    )
    def _sc_spmm(u_hbm, src_hbm, dst_hbm, zeros_hbm, out_hbm,
                 agg_sp, src_v, dst_v, gbufs, gsems, ssems):
        c = lax.axis_index("c")
        s = lax.axis_index("s")
        rows = pl.ds(s * ROWS_PER_TILE, ROWS_PER_TILE)
        u_c = u_hbm.at[c]
        pltpu.sync_copy(zeros_hbm.at[rows], agg_sp.at[rows])
        pltpu.sync_copy(src_hbm.at[s], src_v)
        pltpu.sync_copy(dst_hbm.at[s], dst_v)
        plsc.subcore_barrier()

        def gather_start(j, b):
            pltpu.async_copy(u_c.at[src_v.at[j]], gbufs[b], gsems[b])

        def gather_wait(b):
            pltpu.make_async_copy(u_c.at[src_v.at[0]], gbufs[b], gsems[b]).wait()

        def scatter_start(j, b):
            pltpu.async_copy(gbufs[b], agg_sp.at[dst_v.at[j]], ssems[b], add=True)

        def scatter_wait(b):
            pltpu.make_async_copy(gbufs[b], agg_sp.at[pl.ds(0, CHUNK)],
                                  ssems[b]).wait()

        for b in range(NBUF - 1):
            gather_start(b, b)

        def body(g, carry):
            for b in range(NBUF):
                i = g * NBUF + b
                @pl.when(i >= 1)
                def _():
                    scatter_wait((b - 1) % NBUF)
                @pl.when(i < NCHUNKS - (NBUF - 1))
                def _():
                    gather_start(i + NBUF - 1, (b - 1) % NBUF)
                gather_wait(b)
                scatter_start(i, b)
            return carry

        lax.fori_loop(0, NCHUNKS // NBUF, body, 0)
        scatter_wait((NCHUNKS - 1) % NBUF)
        plsc.subcore_barrier()
        pltpu.sync_copy(agg_sp.at[rows], out_hbm.at[c, rows])

    return _sc_spmm


_sc_spmm_64 = _make_sc_spmm(64)
_sc_spmm_32 = _make_sc_spmm(32)


_BLK = 1024
_GRID = N_PAD // _BLK


def _norm_col(deg_ref):
    return lax.rsqrt(jnp.clip(deg_ref[:, :1], 1.0, None))


def _tc1_body(x_ref, w1_ref, dego_ref, out_ref):
    ns = _norm_col(dego_ref)
    u = jnp.dot(x_ref[...], w1_ref[...], preferred_element_type=jnp.float32) * ns
    out_ref[0] = u[:, :64]
    out_ref[1] = u[:, 64:]


def _tc2_body(agg_ref, degi_ref, dego_ref, w2_ref, b1_ref, out_ref):
    nd = _norm_col(degi_ref)
    ns = _norm_col(dego_ref)
    agg = jnp.concatenate([agg_ref[0], agg_ref[1]], axis=1)
    h = jax.nn.relu(agg * nd + b1_ref[...])
    u2 = jnp.dot(h, w2_ref[...], preferred_element_type=jnp.float32) * ns
    out_ref[0] = u2[:, :32]
    out_ref[1] = u2[:, 32:]


def _tc3_body(agg_ref, degi_ref, b2_ref, out_ref):
    nd = _norm_col(degi_ref)
    agg = jnp.concatenate([agg_ref[0], agg_ref[1]], axis=1)
    out_ref[...] = jax.nn.relu(agg * nd + b2_ref[...])


_tc1 = pl.pallas_call(
    _tc1_body,
    grid=(_GRID,),
    in_specs=[
        pl.BlockSpec((_BLK, D_IN), lambda i: (i, 0)),
        pl.BlockSpec((D_IN, D_IN), lambda i: (0, 0)),
        pl.BlockSpec((_BLK, 16), lambda i: (i, 0)),
    ],
    out_specs=pl.BlockSpec((NC, _BLK, 64), lambda i: (0, i, 0)),
    out_shape=jax.ShapeDtypeStruct((NC, N_PAD, 64), jnp.float32),
)

_tc2 = pl.pallas_call(
    _tc2_body,
    grid=(_GRID,),
    in_specs=[
        pl.BlockSpec((NC, _BLK, 64), lambda i: (0, i, 0)),
        pl.BlockSpec((_BLK, 16), lambda i: (i, 0)),
        pl.BlockSpec((_BLK, 16), lambda i: (i, 0)),
        pl.BlockSpec((D_IN, N_CLASSES), lambda i: (0, 0)),
        pl.BlockSpec((1, D_IN), lambda i: (0, 0)),
    ],
    out_specs=pl.BlockSpec((NC, _BLK, 32), lambda i: (0, i, 0)),
    out_shape=jax.ShapeDtypeStruct((NC, N_PAD, 32), jnp.float32),
)

_tc3 = pl.pallas_call(
    _tc3_body,
    grid=(_GRID,),
    in_specs=[
        pl.BlockSpec((NC, _BLK, 32), lambda i: (0, i, 0)),
        pl.BlockSpec((_BLK, 16), lambda i: (i, 0)),
        pl.BlockSpec((1, N_CLASSES), lambda i: (0, 0)),
    ],
    out_specs=pl.BlockSpec((_BLK, N_CLASSES), lambda i: (i, 0)),
    out_shape=jax.ShapeDtypeStruct((N_PAD, N_CLASSES), jnp.float32),
)


def kernel(x, edge_index, W1, b1, W2, b2):
    edge_index = edge_index.astype(jnp.int32)
    edges_r = edge_index.reshape(NC, NS, NCHUNKS, CHUNK)
    src_r = edges_r[0]
    dst_r = edges_r[1]

    x_pad = jnp.pad(x, ((0, N_PAD - N_NODES), (0, 0)))
    ones16 = jnp.ones((CHUNK, 16), jnp.float32)
    z16 = jnp.zeros((N_PAD, 16), jnp.float32)
    z64 = jnp.zeros((N_PAD, 64), jnp.float32)
    z32 = jnp.zeros((N_PAD, 32), jnp.float32)

    degs = _sc_degrees(edges_r, ones16, z16)
    deg_out = degs[0]
    deg_in = degs[1]

    u1 = _tc1(x_pad, W1, deg_out)
    agg1 = _sc_spmm_64(u1, src_r, dst_r, z64)
    u2 = _tc2(agg1, deg_in, deg_out, W2, b1.reshape(1, -1))
    agg2 = _sc_spmm_32(u2, src_r, dst_r, z32)
    out = _tc3(agg2, deg_in, b2.reshape(1, -1))
    return out[:N_NODES]
